# baseline (speedup 1.0000x reference)
import functools

import jax
import jax.numpy as jnp
from jax import lax
from jax.experimental import pallas as pl
from jax.experimental.pallas import tpu as pltpu
from jax.experimental.pallas import tpu_sc as plsc

VOCAB = 1000000
DIM = 64
BATCH = 16384

_INFO = plsc.get_sparse_core_info()
NC, NS = _INFO.num_cores, _INFO.num_subcores
NW = NC * NS
B_PER_W = BATCH // NW


def _sc_gather(table2, idx2):
    mesh = plsc.VectorSubcoreMesh(core_axis_name="c", subcore_axis_name="s")

    @functools.partial(
        pl.kernel,
        mesh=mesh,
        out_type=jax.ShapeDtypeStruct((BATCH, 2 * DIM), jnp.float32),
        scratch_types=[
            pltpu.VMEM((B_PER_W,), jnp.int32),
            pltpu.VMEM((B_PER_W, 2 * DIM), jnp.float32),
            pltpu.SemaphoreType.DMA,
        ],
    )
    def k(tbl_hbm, idx_hbm, out_hbm, idx_v, rows_v, sem):
        wid = lax.axis_index("s") * NC + lax.axis_index("c")
        base = wid * B_PER_W
        pltpu.sync_copy(idx_hbm.at[pl.ds(base, B_PER_W)], idx_v)
        pltpu.async_copy(tbl_hbm.at[idx_v], rows_v, sem).wait()
        pltpu.sync_copy(rows_v, out_hbm.at[pl.ds(base, B_PER_W)])

    return k(table2, idx2)


def _tc_body(r_ref, p_ref, w_ref, b_ref, o_ref):
    recs = r_ref[...]
    sel = p_ref[...] > 0.5
    x = jnp.where(sel, recs[:, DIM:], recs[:, :DIM])
    y = lax.dot_general(x, w_ref[...], (((1,), (1,)), ((), ())),
                        preferred_element_type=jnp.float32)
    o_ref[...] = y + b_ref[...]


def _tc_linear(recs, par, W, b2):
    blk = 2048
    return pl.pallas_call(
        _tc_body,
        grid=(BATCH // blk,),
        in_specs=[
            pl.BlockSpec((blk, 2 * DIM), lambda i: (i, 0)),
            pl.BlockSpec((blk, 1), lambda i: (i, 0)),
            pl.BlockSpec((DIM, DIM), lambda i: (0, 0)),
            pl.BlockSpec((1, DIM), lambda i: (0, 0)),
        ],
        out_specs=pl.BlockSpec((blk, DIM), lambda i: (i, 0)),
        out_shape=jax.ShapeDtypeStruct((BATCH, DIM), jnp.float32),
    )(recs, par, W, b2)


def kernel(input_doc_id, embedding_table, W, b):
    idx = input_doc_id.astype(jnp.int32)
    table2 = embedding_table.reshape(VOCAB // 2, 2 * DIM)
    recs = _sc_gather(table2, idx >> 1)
    par = (idx & 1).astype(jnp.float32).reshape(BATCH, 1)
    return _tc_linear(recs, par, W, b.reshape(1, DIM))


# speedup vs baseline: 1.0030x; 1.0030x over previous
import functools

import jax
import jax.numpy as jnp
from jax import lax
from jax.experimental import pallas as pl
from jax.experimental.pallas import tpu as pltpu
from jax.experimental.pallas import tpu_sc as plsc

VOCAB = 1000000
DIM = 64
BATCH = 16384

_INFO = plsc.get_sparse_core_info()
NC, NS = _INFO.num_cores, _INFO.num_subcores
NW = NC * NS
B_PER_W = BATCH // NW


def _sc_gather(table2, idx2):
    mesh = plsc.VectorSubcoreMesh(core_axis_name="c", subcore_axis_name="s")

    @functools.partial(
        pl.kernel,
        mesh=mesh,
        out_type=jax.ShapeDtypeStruct((BATCH, 2 * DIM), jnp.float32),
        scratch_types=[
            pltpu.VMEM((B_PER_W,), jnp.int32),
            pltpu.VMEM((B_PER_W, 2 * DIM), jnp.float32),
            pltpu.SemaphoreType.DMA,
        ],
    )
    def k(tbl_hbm, idx_hbm, out_hbm, idx_v, rows_v, sem):
        wid = lax.axis_index("s") * NC + lax.axis_index("c")
        base = wid * B_PER_W
        pltpu.sync_copy(idx_hbm.at[pl.ds(base, B_PER_W)], idx_v)
        pltpu.async_copy(tbl_hbm.at[idx_v], rows_v, sem).wait()
        pltpu.sync_copy(rows_v, out_hbm.at[pl.ds(base, B_PER_W)])

    return k(table2, idx2)


def _tc_body(r_ref, p_ref, w_ref, b_ref, o_ref):
    recs = r_ref[...]
    sel = p_ref[...] > 0.5
    x = jnp.where(sel, recs[:, DIM:], recs[:, :DIM])
    y = lax.dot_general(x, w_ref[...], (((1,), (1,)), ((), ())),
                        preferred_element_type=jnp.float32)
    o_ref[...] = y + b_ref[...]


def _tc_linear(recs, par, W, b2):
    blk = 2048
    return pl.pallas_call(
        _tc_body,
        grid=(BATCH // blk,),
        in_specs=[
            pl.BlockSpec((blk, 2 * DIM), lambda i: (i, 0)),
            pl.BlockSpec((blk, 1), lambda i: (i, 0)),
            pl.BlockSpec((DIM, DIM), lambda i: (0, 0)),
            pl.BlockSpec((1, DIM), lambda i: (0, 0)),
        ],
        out_specs=pl.BlockSpec((blk, DIM), lambda i: (i, 0)),
        out_shape=jax.ShapeDtypeStruct((BATCH, DIM), jnp.float32),
    )(recs, par, W, b2)


def kernel(input_doc_id, embedding_table, W, b):
    idx = input_doc_id.astype(jnp.int32)
    table2 = jnp.reshape(embedding_table, (VOCAB // 2, 2 * DIM))
    recs = _sc_gather(table2, idx >> 1)
    par = (idx & 1).astype(jnp.float32).reshape(BATCH, 1)
    return _tc_linear(recs, par, W, b.reshape(1, DIM))
